# Initial kernel scaffold; baseline (speedup 1.0000x reference)
import jax
import jax.numpy as jnp
from jax.experimental import pallas as pl
from jax.experimental.pallas import tpu as pltpu

N = 10000
E = 160000
C5 = 5000
C6 = 5000
H = 64
NG = 64


def _bn(v, g, b):
    m = v.mean(0)
    s = v.var(0)
    return (v - m) / jnp.sqrt(s + 1e-5) * g + b


def _head_kernel(reps_ref, w1_ref, b1_ref, g1_ref, be1_ref, w2_ref, b2_ref,
                 g2_ref, be2_ref, lw_ref, lb_ref, out_ref):
    reps = reps_ref[...]
    v = jnp.dot(reps, w1_ref[...], preferred_element_type=jnp.float32) + b1_ref[...]
    m = v.mean(0)
    s = ((v - m) ** 2).mean(0)
    v = jax.nn.relu((v - m) / jnp.sqrt(s + 1e-5) * g1_ref[...] + be1_ref[...])
    v = jnp.dot(v, w2_ref[...], preferred_element_type=jnp.float32) + b2_ref[...]
    m = v.mean(0)
    s = ((v - m) ** 2).mean(0)
    v = jax.nn.relu((v - m) / jnp.sqrt(s + 1e-5) * g2_ref[...] + be2_ref[...])
    out_ref[...] = jnp.dot(v, lw_ref[...], preferred_element_type=jnp.float32) + lb_ref[...]


def kernel(x, edge_index, edge_attr, cycle5, cycle6, batch, params):
    src = edge_index[0]
    dst = edge_index[1]
    cyc5p = jnp.pad(cycle5, ((0, 0), (0, 1)))
    cyc_nodes = jnp.concatenate([cyc5p, cycle6], 0)
    maskb = jnp.concatenate([
        jnp.concatenate([jnp.ones((C5, 5), bool), jnp.zeros((C5, 1), bool)], 1),
        jnp.ones((C6, 6), bool)], 0)
    cidx = jnp.where(maskb, cyc_nodes, N)

    def S(e, d):
        return jnp.zeros((N, d), e.dtype).at[src].add(e).at[dst].add(e)
    def G(nf):
        return nf[src] + nf[dst]
    def Sc(c, d):
        return jnp.zeros((N + 1, d), c.dtype).at[cidx.reshape(-1)].add(
            jnp.repeat(c, 6, axis=0))[:N]
    def Gc(nf):
        nfp = jnp.concatenate([nf, jnp.zeros((1, nf.shape[1]), nf.dtype)], 0)
        return nfp[cidx].sum(1)

    p = params
    node_rep = p['node_emb'][x]
    e0 = p['edge_emb'][edge_attr]
    ge0n = G(node_rep)
    se0 = S(e0, H)
    gse0 = G(se0)
    erep = jnp.concatenate([ge0n, gse0], -1)
    erep = jax.nn.relu(erep @ p['edge_mlp1_W'] + p['edge_mlp1_b'])
    erep = jax.nn.relu(erep @ p['edge_mlp2_W'] + p['edge_mlp2_b'])
    cr = Gc(node_rep)
    scr = Sc(cr, H)
    gscr = Gc(scr)
    crep = jnp.concatenate([gscr, cr], -1)
    crep = jax.nn.relu(crep @ p['cycle_mlp1_W'] + p['cycle_mlp1_b'])
    crep = jax.nn.relu(crep @ p['cycle_mlp2_W'] + p['cycle_mlp2_b'])

    for lp in p['layers']:
        h1 = S(erep, H)
        gh1 = G(h1)
        ph = S(gh1, H)
        gp = G(ph)
        W = lp['ee_W1']
        We1 = W[0:64] + W[256:320]
        We2 = W[64:128]
        We3 = W[128:192] + W[192:256]
        pre1 = erep @ We1 + gp @ We2 + gh1 @ We3
        eo1 = jax.nn.relu(_bn(pre1, lp['ee_g1'], lp['ee_be1']))
        eo1 = jax.nn.relu(_bn(eo1 @ lp['ee_W2'], lp['ee_g2'], lp['ee_be2']))
        a_half = Gc(h1)
        q = Sc(a_half, H)
        gq = Gc(q)
        B = lp['ece_W1']
        B0 = B[0:64]
        Bs = B[64:384] + B[384:704]
        mc = gq @ (Bs[0:64] + Bs[64:128]) + a_half @ (Bs[128:192] + Bs[192:256]) + crep @ Bs[256:320]
        mn = Sc(mc, 2 * H)
        gm = G(mn)
        pre2 = erep @ B0 + gm
        eo2 = jax.nn.relu(_bn(pre2, lp['ece_g1'], lp['ece_be1']))
        eo2 = jax.nn.relu(_bn(eo2 @ lp['ece_W2'], lp['ece_g2'], lp['ece_be2']))
        Wc = lp['ecc_W1']
        prec = gq @ (Wc[0:64] + Wc[64:128]) + a_half @ (Wc[128:192] + Wc[192:256]) + crep @ Wc[256:320]
        co = jax.nn.relu(_bn(prec, lp['ecc_g1'], lp['ecc_be1']))
        co = jax.nn.relu(_bn(co @ lp['ecc_W2'], lp['ecc_g2'], lp['ecc_be2']))
        pre3 = jnp.concatenate([eo1, eo2], -1) @ lp['conv_W']
        erep = jax.nn.relu(_bn(pre3, lp['conv_g'], lp['conv_be']))
        crep = co

    n1 = S(erep, H)
    n2 = Sc(crep, H)
    reps = jnp.concatenate([n1, n2], -1)
    sums = jax.ops.segment_sum(reps, batch, num_segments=NG)
    cnt = jax.ops.segment_sum(jnp.ones((N, 1), jnp.float32), batch, num_segments=NG)
    reps = sums / jnp.maximum(cnt, 1.0)

    out = pl.pallas_call(
        _head_kernel,
        out_shape=jax.ShapeDtypeStruct((NG, 1), jnp.float32),
    )(reps, p['final_W1'], p['final_b1'], p['final_g1'], p['final_be1'],
      p['final_W2'], p['final_b2'], p['final_g2'], p['final_be2'],
      p['lin_W'], p['lin_b'])
    return out


# speedup vs baseline: 1.0275x; 1.0275x over previous
import jax
import jax.numpy as jnp
from jax.experimental import pallas as pl
from jax.experimental.pallas import tpu as pltpu

N = 10000
E = 160000
C5 = 5000
C6 = 5000
H = 64
NG = 64


def _bn(v, g, b):
    m = v.mean(0)
    s = v.var(0)
    return (v - m) / jnp.sqrt(s + 1e-5) * g + b


def _head_kernel(reps_ref, w1_ref, b1_ref, g1_ref, be1_ref, w2_ref, b2_ref,
                 g2_ref, be2_ref, lw_ref, lb_ref, out_ref):
    reps = reps_ref[...]
    v = jnp.dot(reps, w1_ref[...], preferred_element_type=jnp.float32) + b1_ref[...]
    m = v.mean(0)
    s = ((v - m) ** 2).mean(0)
    v = jax.nn.relu((v - m) / jnp.sqrt(s + 1e-5) * g1_ref[...] + be1_ref[...])
    v = jnp.dot(v, w2_ref[...], preferred_element_type=jnp.float32) + b2_ref[...]
    m = v.mean(0)
    s = ((v - m) ** 2).mean(0)
    v = jax.nn.relu((v - m) / jnp.sqrt(s + 1e-5) * g2_ref[...] + be2_ref[...])
    out_ref[...] = jnp.dot(v, lw_ref[...], preferred_element_type=jnp.float32) + lb_ref[...]


def kernel(x, edge_index, edge_attr, cycle5, cycle6, batch, params):
    return _kern(x, edge_index, edge_attr, cycle5, cycle6, batch, params)


def _kern(x, edge_index, edge_attr, cycle5, cycle6, batch, params):
    src = edge_index[0]
    dst = edge_index[1]
    cyc5p = jnp.pad(cycle5, ((0, 0), (0, 1)))
    cyc_nodes = jnp.concatenate([cyc5p, cycle6], 0)
    maskb = jnp.concatenate([
        jnp.concatenate([jnp.ones((C5, 5), bool), jnp.zeros((C5, 1), bool)], 1),
        jnp.ones((C6, 6), bool)], 0)
    cidx = jnp.where(maskb, cyc_nodes, N)

    def S(e, d):
        return jnp.zeros((N, d), e.dtype).at[src].add(e).at[dst].add(e)
    def G(nf):
        return nf[src] + nf[dst]
    def Sc(c, d):
        return jnp.zeros((N + 1, d), c.dtype).at[cidx.reshape(-1)].add(
            jnp.repeat(c, 6, axis=0))[:N]
    def Gc(nf):
        nfp = jnp.concatenate([nf, jnp.zeros((1, nf.shape[1]), nf.dtype)], 0)
        return nfp[cidx].sum(1)

    p = params
    node_rep = p['node_emb'][x]
    e0 = p['edge_emb'][edge_attr]
    ge0n = G(node_rep)
    se0 = S(e0, H)
    gse0 = G(se0)
    erep = jnp.concatenate([ge0n, gse0], -1)
    erep = jax.nn.relu(erep @ p['edge_mlp1_W'] + p['edge_mlp1_b'])
    erep = jax.nn.relu(erep @ p['edge_mlp2_W'] + p['edge_mlp2_b'])
    cr = Gc(node_rep)
    scr = Sc(cr, H)
    gscr = Gc(scr)
    crep = jnp.concatenate([gscr, cr], -1)
    crep = jax.nn.relu(crep @ p['cycle_mlp1_W'] + p['cycle_mlp1_b'])
    crep = jax.nn.relu(crep @ p['cycle_mlp2_W'] + p['cycle_mlp2_b'])

    def g_e2e(e):
        return jnp.concatenate([G(S(e, e.shape[1])), e], -1)
    def n2c_full(nf):
        nfp = jnp.concatenate([nf, jnp.zeros((1, nf.shape[1]), nf.dtype)], 0)
        return nfp[cidx].sum(1)
    def g_e2c(e):
        a = n2c_full(S(e, e.shape[1]))
        return jnp.concatenate([a, a], -1)
    def g_c2c(c):
        return jnp.concatenate([n2c_full(Sc(c, c.shape[1])), c], -1)
    def g_c2e(c):
        a = G(Sc(c, c.shape[1]))
        return jnp.concatenate([a, a], -1)

    for lp in p['layers']:
        ee = g_e2e(g_e2e(erep))
        eo1 = jax.nn.relu(_bn(jnp.concatenate([erep, ee], -1) @ lp['ee_W1'], lp['ee_g1'], lp['ee_be1']))
        eo1 = jax.nn.relu(_bn(eo1 @ lp['ee_W2'], lp['ee_g2'], lp['ee_be2']))
        e2c = g_c2c(g_e2c(erep))
        cycle_new = jnp.concatenate([e2c, crep], -1)
        c2e = g_c2e(cycle_new)
        eo2 = jax.nn.relu(_bn(jnp.concatenate([erep, c2e], -1) @ lp['ece_W1'], lp['ece_g1'], lp['ece_be1']))
        eo2 = jax.nn.relu(_bn(eo2 @ lp['ece_W2'], lp['ece_g2'], lp['ece_be2']))
        co = jax.nn.relu(_bn(cycle_new @ lp['ecc_W1'], lp['ecc_g1'], lp['ecc_be1']))
        co = jax.nn.relu(_bn(co @ lp['ecc_W2'], lp['ecc_g2'], lp['ecc_be2']))
        erep = jax.nn.relu(_bn(jnp.concatenate([eo1, eo2], -1) @ lp['conv_W'], lp['conv_g'], lp['conv_be']))
        crep = co

    n1 = S(erep, H)
    n2 = Sc(crep, H)
    reps = jnp.concatenate([n1, n2], -1)
    sums = jax.ops.segment_sum(reps, batch, num_segments=NG)
    cnt = jax.ops.segment_sum(jnp.ones((N, 1), jnp.float32), batch, num_segments=NG)
    reps = sums / jnp.maximum(cnt, 1.0)

    out = pl.pallas_call(
        _head_kernel,
        out_shape=jax.ShapeDtypeStruct((NG, 1), jnp.float32),
    )(reps, p['final_W1'], p['final_b1'], p['final_g1'], p['final_be1'],
      p['final_W2'], p['final_b2'], p['final_g2'], p['final_be2'],
      p['lin_W'], p['lin_b'])
    return out


# speedup vs baseline: 1.1932x; 1.1613x over previous
import functools
import jax
import jax.numpy as jnp
from jax import lax
from jax.experimental import pallas as pl
from jax.experimental.pallas import tpu as pltpu
from jax.experimental.pallas import tpu_sc as plsc

N = 10000
E = 160000
C5 = 5000
C6 = 5000
C = C5 + C6
H = 64
NG = 64

NC = 2
NS = 16
NW = NC * NS
GR = 128

EP = 163840
CP = 12288
NP = 12288
GE = EP // GR
GC = CP // GR
GN = NP // GR
DUM = N


def _sc_mesh():
    return plsc.VectorSubcoreMesh(core_axis_name="c", subcore_axis_name="s",
                                  num_cores=NC, num_subcores=NS)


@functools.cache
def _make_gather(k, ngroups, d):
    gpw = ngroups // NW
    nsl = d // 16

    def body(table, idx, out, idx_v, rows_v, sem):
        wid = lax.axis_index("s") * NC + lax.axis_index("c")

        def grp(i, carry):
            g = wid * gpw + i
            pltpu.sync_copy(idx.at[g], idx_v)
            descs = [
                pltpu.async_copy(table.at[idx_v.at[j]], rows_v.at[j], sem)
                for j in range(k)
            ]
            for dsc in descs:
                dsc.wait()
            if k > 1:
                def row(r, c2):
                    for cc in range(nsl):
                        sl = pl.ds(cc * 16, 16)
                        acc = rows_v[0, r, sl]
                        for j in range(1, k):
                            acc = acc + rows_v[j, r, sl]
                        rows_v[0, r, sl] = acc
                    return c2
                lax.fori_loop(0, GR, row, 0, unroll=2)
            pltpu.sync_copy(rows_v.at[0], out.at[pl.ds(g * GR, GR)])
            return carry

        lax.fori_loop(0, gpw, grp, 0)

    return pl.kernel(
        body,
        out_type=jax.ShapeDtypeStruct((ngroups * GR, d), jnp.float32),
        mesh=_sc_mesh(),
        compiler_params=pltpu.CompilerParams(use_tc_tiling_on_sc=False),
        scratch_types=[
            pltpu.VMEM((k, GR), jnp.int32),
            pltpu.VMEM((k, GR, d), jnp.float32),
            pltpu.SemaphoreType.DMA,
        ],
    )


@functools.cache
def _make_scatter(k, ngroups, d):
    gpw = ngroups // NW
    stripe = NP // NS
    nsl = d // 16

    def body(rows, idx, out, idx_v, rows_v, acc):
        cid = lax.axis_index("c")
        sid = lax.axis_index("s")
        wid = sid * NC + cid

        def zr(r, c2):
            for cc in range(nsl):
                rows_v[r, pl.ds(cc * 16, 16)] = jnp.zeros((16,), jnp.float32)
            return c2
        lax.fori_loop(0, GR, zr, 0)
        for t in range(stripe // GR):
            pltpu.sync_copy(rows_v, acc.at[pl.ds(sid * stripe + t * GR, GR)])
        plsc.subcore_barrier()

        def grp(i, carry):
            g = wid * gpw + i
            pltpu.sync_copy(rows.at[pl.ds(g * GR, GR)], rows_v)
            pltpu.sync_copy(idx.at[g], idx_v)
            for j in range(k):
                pltpu.sync_copy(rows_v, acc.at[idx_v.at[j]], add=True)
            return carry
        lax.fori_loop(0, gpw, grp, 0)

        plsc.subcore_barrier()
        pltpu.sync_copy(acc.at[pl.ds(sid * stripe, stripe)],
                        out.at[cid, pl.ds(sid * stripe, stripe)])

    return pl.kernel(
        body,
        out_type=jax.ShapeDtypeStruct((NC, NP, d), jnp.float32),
        mesh=_sc_mesh(),
        compiler_params=pltpu.CompilerParams(use_tc_tiling_on_sc=False),
        scratch_types=[
            pltpu.VMEM((k, GR), jnp.int32),
            pltpu.VMEM((GR, d), jnp.float32),
            pltpu.VMEM_SHARED((NP, d), jnp.float32),
        ],
    )


def _dot(a, b):
    return jax.lax.dot_general(a, b, (((1,), (0,)), ((), ())),
                               preferred_element_type=jnp.float32)


def _combine_body(p_ref, o_ref):
    pid = pl.program_id(0)
    blk = o_ref.shape[0]
    rows = pid * blk + lax.broadcasted_iota(jnp.int32, (blk, 1), 0)
    s = p_ref[0] + p_ref[1]
    o_ref[...] = jnp.where(rows < N, s, 0.0)


def _combine(partials, d):
    blk = 512
    return pl.pallas_call(
        _combine_body,
        grid=(NP // blk,),
        in_specs=[pl.BlockSpec((NC, blk, d), lambda i: (0, i, 0))],
        out_specs=pl.BlockSpec((blk, d), lambda i: (i, 0)),
        out_shape=jax.ShapeDtypeStruct((NP, d), jnp.float32),
    )(partials)


def _stats_rows(acc01):
    return jnp.concatenate([acc01, jnp.zeros((6, acc01.shape[1]), jnp.float32)], 0)


def _bn_apply(v, stats, g, be, valid):
    m = stats[0:1, :]
    var = stats[1:2, :] / valid
    return (v - m) / jnp.sqrt(var + 1e-5) * g + be


def _mask(pid, blk, valid):
    rows = pid * blk + lax.broadcasted_iota(jnp.int32, (blk, 1), 0)
    return (rows < valid).astype(jnp.float32)


def _stats_merge(acc, y, pid, blk, valid, row0):
    nb = jnp.clip(valid - pid * blk, 0, blk).astype(jnp.float32)
    na = jnp.clip(pid * blk, 0, valid).astype(jnp.float32)
    msk = _mask(pid, blk, valid)
    my = y * msk
    mb = my.sum(0, keepdims=True) / jnp.maximum(nb, 1.0)
    m2b = (((y - mb) * msk) ** 2).sum(0, keepdims=True)
    tot = jnp.maximum(na + nb, 1.0)
    delta = mb - acc[row0:row0 + 1, :]
    acc[row0:row0 + 1, :] += delta * (nb / tot)
    acc[row0 + 1:row0 + 2, :] += m2b + delta * delta * (na * nb / tot)


def _mlp2_body(x1_ref, x2_ref, w1_ref, b1_ref, w2_ref, b2_ref, o_ref):
    v = _dot(jnp.concatenate([x1_ref[...], x2_ref[...]], axis=1),
             w1_ref[...]) + b1_ref[...]
    v = jax.nn.relu(v)
    v = _dot(v, w2_ref[...]) + b2_ref[...]
    o_ref[...] = jax.nn.relu(v)


def _mlp2(x1, x2, w1, b1, w2, b2, nrows, blk):
    full = lambda s: pl.BlockSpec(s, lambda i: tuple(0 for _ in s))
    return pl.pallas_call(
        _mlp2_body,
        grid=(nrows // blk,),
        in_specs=[
            pl.BlockSpec((blk, 64), lambda i: (i, 0)),
            pl.BlockSpec((blk, 64), lambda i: (i, 0)),
            full((128, 128)), full((1, 128)), full((128, 64)), full((1, 64)),
        ],
        out_specs=pl.BlockSpec((blk, 64), lambda i: (i, 0)),
        out_shape=jax.ShapeDtypeStruct((nrows, 64), jnp.float32),
    )(x1, x2, w1, b1, w2, b2)


def _k1_body(e_ref, gh1_ref, gp_ref, gu_ref, gq_ref, gw_ref, w1_ref, w2_ref,
             v1_ref, v2_ref, s1_ref, s2_ref, acc):
    pid = pl.program_id(0)
    ng = pl.num_programs(0)
    e = e_ref[...]
    gh1 = gh1_ref[...]
    gp = gp_ref[...]
    gu = gu_ref[...]
    gq = gq_ref[...]
    gw = gw_ref[...]
    cat1 = jnp.concatenate([e, gp, gh1, gh1, e], axis=1)
    p1 = _dot(cat1, w1_ref[...])
    cat2 = jnp.concatenate([e, gu, gu, gq, gq, gw, gu, gu, gq, gq, gw], axis=1)
    p2 = _dot(cat2, w2_ref[...])
    v1_ref[...] = p1
    v2_ref[...] = p2

    @pl.when(pid == 0)
    def _():
        acc[...] = jnp.zeros_like(acc)

    _stats_merge(acc, p1, pid, p1.shape[0], E, 0)
    _stats_merge(acc, p2, pid, p2.shape[0], E, 2)

    @pl.when(pid == ng - 1)
    def _():
        s1_ref[...] = _stats_rows(acc[0:2, :])
        s2_ref[...] = _stats_rows(acc[2:4, :])


def _k1(e, gh1, gp, gu, gq, gw, w1, w2):
    blk = 1024
    full = lambda s: pl.BlockSpec(s, lambda i: tuple(0 for _ in s))
    eb = lambda: pl.BlockSpec((blk, 64), lambda i: (i, 0))
    return pl.pallas_call(
        _k1_body,
        grid=(EP // blk,),
        in_specs=[eb(), eb(), eb(), eb(), eb(), eb(),
                  full((320, 128)), full((704, 128))],
        out_specs=[pl.BlockSpec((blk, 128), lambda i: (i, 0)),
                   pl.BlockSpec((blk, 128), lambda i: (i, 0)),
                   full((8, 128)), full((8, 128))],
        out_shape=[jax.ShapeDtypeStruct((EP, 128), jnp.float32),
                   jax.ShapeDtypeStruct((EP, 128), jnp.float32),
                   jax.ShapeDtypeStruct((8, 128), jnp.float32),
                   jax.ShapeDtypeStruct((8, 128), jnp.float32)],
        scratch_shapes=[pltpu.VMEM((8, 128), jnp.float32)],
    )(e, gh1, gp, gu, gq, gw, w1, w2)


def _bnd_body(x_ref, s_ref, g_ref, be_ref, w_ref, o_ref, so_ref, acc, *, valid):
    pid = pl.program_id(0)
    ng = pl.num_programs(0)
    v = _bn_apply(x_ref[...], s_ref[...], g_ref[...], be_ref[...], valid)
    y = _dot(jax.nn.relu(v), w_ref[...])
    o_ref[...] = y

    @pl.when(pid == 0)
    def _():
        acc[...] = jnp.zeros_like(acc)

    _stats_merge(acc, y, pid, y.shape[0], valid, 0)

    @pl.when(pid == ng - 1)
    def _():
        so_ref[...] = _stats_rows(acc[0:2, :])


def _bnd(x, s, g, be, w, nrows, valid, din, dout, blk):
    full = lambda sh: pl.BlockSpec(sh, lambda i: tuple(0 for _ in sh))
    return pl.pallas_call(
        functools.partial(_bnd_body, valid=valid),
        grid=(nrows // blk,),
        in_specs=[pl.BlockSpec((blk, din), lambda i: (i, 0)),
                  full((8, din)), full((1, din)), full((1, din)),
                  full((din, dout))],
        out_specs=[pl.BlockSpec((blk, dout), lambda i: (i, 0)),
                   full((8, dout))],
        out_shape=[jax.ShapeDtypeStruct((nrows, dout), jnp.float32),
                   jax.ShapeDtypeStruct((8, dout), jnp.float32)],
        scratch_shapes=[pltpu.VMEM((8, dout), jnp.float32)],
    )(x, s, g, be, w)


def _k3_body(x1_ref, s1_ref, g1_ref, be1_ref, x2_ref, s2_ref, g2_ref, be2_ref,
             w_ref, o_ref, so_ref, acc, *, valid):
    pid = pl.program_id(0)
    ng = pl.num_programs(0)
    eo1 = jax.nn.relu(_bn_apply(x1_ref[...], s1_ref[...], g1_ref[...], be1_ref[...], valid))
    eo2 = jax.nn.relu(_bn_apply(x2_ref[...], s2_ref[...], g2_ref[...], be2_ref[...], valid))
    y = _dot(jnp.concatenate([eo1, eo2], axis=1), w_ref[...])
    o_ref[...] = y

    @pl.when(pid == 0)
    def _():
        acc[...] = jnp.zeros_like(acc)

    _stats_merge(acc, y, pid, y.shape[0], valid, 0)

    @pl.when(pid == ng - 1)
    def _():
        so_ref[...] = _stats_rows(acc[0:2, :])


def _k3(x1, s1, g1, be1, x2, s2, g2, be2, w, nrows, valid, blk):
    full = lambda sh: pl.BlockSpec(sh, lambda i: tuple(0 for _ in sh))
    xb = lambda: pl.BlockSpec((blk, 64), lambda i: (i, 0))
    return pl.pallas_call(
        functools.partial(_k3_body, valid=valid),
        grid=(nrows // blk,),
        in_specs=[xb(), full((8, 64)), full((1, 64)), full((1, 64)),
                  xb(), full((8, 64)), full((1, 64)), full((1, 64)),
                  full((128, 64))],
        out_specs=[xb(), full((8, 64))],
        out_shape=[jax.ShapeDtypeStruct((nrows, 64), jnp.float32),
                   jax.ShapeDtypeStruct((8, 64), jnp.float32)],
        scratch_shapes=[pltpu.VMEM((8, 64), jnp.float32)],
    )(x1, s1, g1, be1, x2, s2, g2, be2, w)


def _k4_body(x_ref, s_ref, g_ref, be_ref, o_ref, *, valid):
    o_ref[...] = jax.nn.relu(
        _bn_apply(x_ref[...], s_ref[...], g_ref[...], be_ref[...], valid))


def _k4(x, s, g, be, nrows, valid, d, blk):
    full = lambda sh: pl.BlockSpec(sh, lambda i: tuple(0 for _ in sh))
    return pl.pallas_call(
        functools.partial(_k4_body, valid=valid),
        grid=(nrows // blk,),
        in_specs=[pl.BlockSpec((blk, d), lambda i: (i, 0)),
                  full((8, d)), full((1, d)), full((1, d))],
        out_specs=pl.BlockSpec((blk, d), lambda i: (i, 0)),
        out_shape=jax.ShapeDtypeStruct((nrows, d), jnp.float32),
    )(x, s, g, be)


def _kc1_body(gq_ref, a_ref, cr_ref, w_ref, o_ref, so_ref, acc):
    pid = pl.program_id(0)
    ng = pl.num_programs(0)
    gq = gq_ref[...]
    a = a_ref[...]
    cr = cr_ref[...]
    y = _dot(jnp.concatenate([gq, gq, a, a, cr], axis=1), w_ref[...])
    o_ref[...] = y

    @pl.when(pid == 0)
    def _():
        acc[...] = jnp.zeros_like(acc)

    _stats_merge(acc, y, pid, y.shape[0], C, 0)

    @pl.when(pid == ng - 1)
    def _():
        so_ref[...] = _stats_rows(acc[0:2, :])


def _kc1(gq, a, cr, w):
    blk = 512
    full = lambda sh: pl.BlockSpec(sh, lambda i: tuple(0 for _ in sh))
    xb = lambda: pl.BlockSpec((blk, 64), lambda i: (i, 0))
    return pl.pallas_call(
        _kc1_body,
        grid=(CP // blk,),
        in_specs=[xb(), xb(), xb(), full((320, 128))],
        out_specs=[pl.BlockSpec((blk, 128), lambda i: (i, 0)), full((8, 128))],
        out_shape=[jax.ShapeDtypeStruct((CP, 128), jnp.float32),
                   jax.ShapeDtypeStruct((8, 128), jnp.float32)],
        scratch_shapes=[pltpu.VMEM((8, 128), jnp.float32)],
    )(gq, a, cr, w)


def _tail_body(n1_ref, n2_ref, oh_ref, fw1_ref, fb1_ref, fg1_ref, fbe1_ref,
               fw2_ref, fb2_ref, fg2_ref, fbe2_ref, lw_ref, lb_ref,
               o_ref, acc, cnt):
    pid = pl.program_id(0)
    ng = pl.num_programs(0)
    r1 = n1_ref[0] + n1_ref[1]
    r2 = n2_ref[0] + n2_ref[1]
    oh = oh_ref[...]

    @pl.when(pid == 0)
    def _():
        acc[...] = jnp.zeros_like(acc)
        cnt[...] = jnp.zeros_like(cnt)

    hi = jax.lax.Precision.HIGHEST
    d1 = jax.lax.dot_general(oh, r1, (((0,), (0,)), ((), ())), precision=hi,
                             preferred_element_type=jnp.float32)
    d2 = jax.lax.dot_general(oh, r2, (((0,), (0,)), ((), ())), precision=hi,
                             preferred_element_type=jnp.float32)
    acc[:, 0:64] += d1
    acc[:, 64:128] += d2
    cnt[...] += oh.sum(0, keepdims=True)

    @pl.when(pid == ng - 1)
    def _():
        reps = acc[...] / jnp.maximum(cnt[...], 1.0).reshape(NG, 1)
        v = _dot(reps, fw1_ref[...]) + fb1_ref[...]
        m = v.mean(0, keepdims=True)
        s = ((v - m) ** 2).mean(0, keepdims=True)
        v = jax.nn.relu((v - m) / jnp.sqrt(s + 1e-5) * fg1_ref[...] + fbe1_ref[...])
        v = _dot(v, fw2_ref[...]) + fb2_ref[...]
        m = v.mean(0, keepdims=True)
        s = ((v - m) ** 2).mean(0, keepdims=True)
        v = jax.nn.relu((v - m) / jnp.sqrt(s + 1e-5) * fg2_ref[...] + fbe2_ref[...])
        o_ref[...] = _dot(v, lw_ref[...]) + lb_ref[...]


def _tail(n1p, n2p, oh, p):
    blk = 1024
    full = lambda sh: pl.BlockSpec(sh, lambda i: tuple(0 for _ in sh))
    return pl.pallas_call(
        _tail_body,
        grid=(NP // blk,),
        in_specs=[pl.BlockSpec((NC, blk, 64), lambda i: (0, i, 0)),
                  pl.BlockSpec((NC, blk, 64), lambda i: (0, i, 0)),
                  pl.BlockSpec((blk, NG), lambda i: (i, 0)),
                  full((128, 64)), full((1, 64)), full((1, 64)), full((1, 64)),
                  full((64, 64)), full((1, 64)), full((1, 64)), full((1, 64)),
                  full((64, 1)), full((1, 1))],
        out_specs=full((NG, 1)),
        out_shape=jax.ShapeDtypeStruct((NG, 1), jnp.float32),
        scratch_shapes=[pltpu.VMEM((NG, 128), jnp.float32),
                        pltpu.VMEM((1, NG), jnp.float32)],
    )(n1p, n2p, oh,
      p['final_W1'], p['final_b1'].reshape(1, -1), p['final_g1'].reshape(1, -1),
      p['final_be1'].reshape(1, -1),
      p['final_W2'], p['final_b2'].reshape(1, -1), p['final_g2'].reshape(1, -1),
      p['final_be2'].reshape(1, -1),
      p['lin_W'], p['lin_b'].reshape(1, 1))


def _pad_rows(arr, rows, value):
    return jnp.concatenate(
        [arr, jnp.full((rows - arr.shape[0],) + arr.shape[1:], value, arr.dtype)], 0)


def _group_idx(idx_kr, ngroups):
    k = idx_kr.shape[0]
    return idx_kr.reshape(k, ngroups, GR).transpose(1, 0, 2)


def kernel(x, edge_index, edge_attr, cycle5, cycle6, batch, params):
    p = params
    src = edge_index[0]
    dst = edge_index[1]

    e_idx = _group_idx(jnp.stack([
        _pad_rows(src, EP, DUM), _pad_rows(dst, EP, DUM)]), GE)

    cyc5p = jnp.pad(cycle5, ((0, 0), (0, 1)))
    cyc_nodes = jnp.concatenate([cyc5p, cycle6], 0)
    maskb = jnp.concatenate([
        jnp.concatenate([jnp.ones((C5, 5), bool), jnp.zeros((C5, 1), bool)], 1),
        jnp.ones((C6, 6), bool)], 0)
    cidx = jnp.where(maskb, cyc_nodes, DUM)
    c_idx = _group_idx(_pad_rows(cidx, CP, DUM).T, GC)

    x_idx = _group_idx(_pad_rows(x, NP, 22)[None], GN)
    ea_idx = _group_idx(_pad_rows(edge_attr, EP, 4)[None], GE)

    node_emb = _pad_rows(p['node_emb'], 24, 0.0)
    edge_emb = _pad_rows(p['edge_emb'], 8, 0.0)

    oh = (batch[:, None] == jnp.arange(NG)[None, :]).astype(jnp.float32)
    oh = _pad_rows(oh, NP, 0.0)

    r2 = lambda a: a.reshape(1, -1)

    ge2 = _make_gather(2, GE, 64)
    gc6 = _make_gather(6, GC, 64)
    se2 = _make_scatter(2, GE, 64)
    sc6 = _make_scatter(6, GC, 64)

    node_rep = _make_gather(1, GN, 64)(node_emb, x_idx)
    e0 = _make_gather(1, GE, 64)(edge_emb, ea_idx)

    ge0n = ge2(node_rep, e_idx)
    se0 = _combine(se2(e0, e_idx), 64)
    gse0 = ge2(se0, e_idx)
    erep = _mlp2(ge0n, gse0, p['edge_mlp1_W'], r2(p['edge_mlp1_b']),
                 p['edge_mlp2_W'], r2(p['edge_mlp2_b']), EP, 1024)

    cr = gc6(node_rep, c_idx)
    scr = _combine(sc6(cr, c_idx), 64)
    gscr = gc6(scr, c_idx)
    crep = _mlp2(gscr, cr, p['cycle_mlp1_W'], r2(p['cycle_mlp1_b']),
                 p['cycle_mlp2_W'], r2(p['cycle_mlp2_b']), CP, 512)

    for lp in p['layers']:
        h1 = _combine(se2(erep, e_idx), 64)
        gh1 = ge2(h1, e_idx)
        ph = _combine(se2(gh1, e_idx), 64)
        gp = ge2(ph, e_idx)

        a = gc6(h1, c_idx)
        q = _combine(sc6(a, c_idx), 64)
        gq = gc6(q, c_idx)
        u = _combine(sc6(gq, c_idx), 64)
        w = _combine(sc6(crep, c_idx), 64)

        gu = ge2(u, e_idx)
        gqe = ge2(q, e_idx)
        gw = ge2(w, e_idx)

        v1, v2, s1, s2 = _k1(erep, gh1, gp, gu, gqe, gw,
                             lp['ee_W1'], lp['ece_W1'])
        w1, sw1 = _bnd(v1, s1, r2(lp['ee_g1']), r2(lp['ee_be1']),
                       lp['ee_W2'], EP, E, 128, 64, 1024)
        w2, sw2 = _bnd(v2, s2, r2(lp['ece_g1']), r2(lp['ece_be1']),
                       lp['ece_W2'], EP, E, 128, 64, 1024)
        pre3, s3 = _k3(w1, sw1, r2(lp['ee_g2']), r2(lp['ee_be2']),
                       w2, sw2, r2(lp['ece_g2']), r2(lp['ece_be2']),
                       lp['conv_W'], EP, E, 1024)
        erep = _k4(pre3, s3, r2(lp['conv_g']), r2(lp['conv_be']), EP, E, 64, 1024)

        prec, sc1 = _kc1(gq, a, crep, lp['ecc_W1'])
        cov, sc2 = _bnd(prec, sc1, r2(lp['ecc_g1']), r2(lp['ecc_be1']),
                        lp['ecc_W2'], CP, C, 128, 64, 512)
        crep = _k4(cov, sc2, r2(lp['ecc_g2']), r2(lp['ecc_be2']), CP, C, 64, 512)

    n1p = se2(erep, e_idx)
    n2p = sc6(crep, c_idx)
    return _tail(n1p, n2p, oh, p)
